# baseline (speedup 1.0000x reference)
import functools

import jax
import jax.numpy as jnp
from jax import lax
from jax.experimental import pallas as pl
from jax.experimental.pallas import tpu as pltpu
from jax.experimental.pallas import tpu_sc as plsc

N = 10000
NP = 10240
F = 128
HID = 64
HEADS = 4
E_TOT = 330000
C = 128
E_PAD = 335872
BM = 1024


def _proj_body(x_ref, w_ref, b_ref, o_ref):
    o_ref[...] = (jnp.dot(x_ref[...], w_ref[0],
                          preferred_element_type=jnp.float32) + b_ref[0])[None]


def _proj(xp, Wbig, bbig):
    return pl.pallas_call(
        _proj_body,
        grid=(9, NP // BM),
        in_specs=[
            pl.BlockSpec((BM, F), lambda j, m: (m, 0)),
            pl.BlockSpec((1, F, HID), lambda j, m: (j, 0, 0)),
            pl.BlockSpec((1, 1, HID), lambda j, m: (j, 0, 0)),
        ],
        out_specs=pl.BlockSpec((1, BM, HID), lambda j, m: (j, m, 0)),
        out_shape=jax.ShapeDtypeStruct((9, NP, HID), jnp.float32),
    )(xp, Wbig, bbig)


def _make_edge_kernel(n_pass, split_edges, heads_tab, interpret=False):
    ACC = n_pass * NP
    ept = E_PAD // (32 if split_edges else 16)
    n_chunks = ept // C
    wrows = ACC // 16
    mesh = plsc.VectorSubcoreMesh(core_axis_name="c", subcore_axis_name="s")

    @functools.partial(
        pl.kernel,
        out_type=(jax.ShapeDtypeStruct((2 * ACC, HID), jnp.float32),
                  jax.ShapeDtypeStruct((2 * ACC, 16), jnp.float32)),
        mesh=mesh,
        scratch_types=[
            pltpu.VMEM_SHARED((ACC, HID), jnp.float32),
            pltpu.VMEM_SHARED((ACC, 16), jnp.float32),
            pltpu.VMEM((heads_tab, HID), jnp.float32),
            pltpu.VMEM((C,), jnp.int32),
            pltpu.VMEM((C,), jnp.int32),
            pltpu.VMEM((C,), jnp.int32),
            pltpu.VMEM((C,), jnp.int32),
            pltpu.VMEM((C,), jnp.int32),
            pltpu.VMEM((C, HID), jnp.float32),
            pltpu.VMEM((C, HID), jnp.float32),
            pltpu.VMEM((C, HID), jnp.float32),
            pltpu.VMEM((C, 16), jnp.float32),
            pltpu.SemaphoreType.DMA,
            pltpu.SemaphoreType.DMA,
        ],
        compiler_params=pltpu.CompilerParams(needs_layout_passes=False,
                                             use_tc_tiling_on_sc=False),
        interpret=interpret,
    )
    def k(xl, xr, srcE, dstE, attH, zv, zd, outv, outd,
          accv, accd, att_vm, src_i, dst_i, srcg, dstg, dsta,
          xls, xrd, vals, exb, sem1, sem2):
        c = lax.axis_index("c")
        s = lax.axis_index("s")
        pltpu.sync_copy(zv.at[pl.ds(s * wrows, wrows)],
                        accv.at[pl.ds(s * wrows, wrows)])
        pltpu.sync_copy(zd.at[pl.ds(s * wrows, wrows)],
                        accd.at[pl.ds(s * wrows, wrows)])
        pltpu.sync_copy(attH, att_vm)
        plsc.subcore_barrier()
        lane = lax.iota(jnp.int32, 16)
        mask0 = lane == 0

        for p in range(n_pass):
            if split_edges:
                tile_base = (c * 16 + s) * ept
                g_off = jnp.int32(0)
                a_off = jnp.int32(0)
                h_idx = 0
            else:
                tile_base = s * ept
                h_idx = 2 * c + p
                g_off = h_idx * NP
                a_off = jnp.int32(p * NP)
            att_q = [att_vm[h_idx, pl.ds(q * 16, 16)] for q in range(HID // 16)]

            def chunk(kk, _):
                eb = tile_base + kk * C
                pltpu.sync_copy(srcE.at[pl.ds(eb, C)], src_i)
                pltpu.sync_copy(dstE.at[pl.ds(eb, C)], dst_i)

                def shift(i, _):
                    sl = pl.ds(i * 16, 16)
                    srcg[sl] = src_i[sl] + g_off
                    dstg[sl] = dst_i[sl] + g_off
                    dsta[sl] = dst_i[sl] + a_off
                    return 0
                lax.fori_loop(0, C // 16, shift, 0)

                cp1 = pltpu.async_copy(xl.at[srcg], xls, sem1)
                cp2 = pltpu.async_copy(xr.at[dstg], xrd, sem2)
                cp1.wait()
                cp2.wait()

                def edge(i, _):
                    for u in range(2):
                        e = i * 2 + u
                        acc = jnp.zeros((16,), jnp.float32)
                        rows = []
                        for q in range(HID // 16):
                            sv = xls[e, pl.ds(q * 16, 16)]
                            rv = xrd[e, pl.ds(q * 16, 16)]
                            t = sv + rv
                            t = jnp.maximum(t, 0.2 * t)
                            acc = acc + t * att_q[q]
                            rows.append(sv)
                        logit = jnp.sum(acc)
                        exv = jnp.exp(jnp.full((16,), logit, jnp.float32))
                        for q in range(HID // 16):
                            vals[e, pl.ds(q * 16, 16)] = rows[q] * exv
                        exb[e, :] = jnp.where(mask0, exv, 0.0)
                    return 0
                lax.fori_loop(0, C // 2, edge, 0)

                pltpu.sync_copy(vals, accv.at[dsta], add=True)
                pltpu.sync_copy(exb, accd.at[dsta], add=True)
                return 0
            lax.fori_loop(0, n_chunks, chunk, 0)

        plsc.subcore_barrier()
        ob = c * ACC + s * wrows
        pltpu.sync_copy(accv.at[pl.ds(s * wrows, wrows)],
                        outv.at[pl.ds(ob, wrows)])
        pltpu.sync_copy(accd.at[pl.ds(s * wrows, wrows)],
                        outd.at[pl.ds(ob, wrows)])

    return k


_edge_l1 = _make_edge_kernel(n_pass=2, split_edges=False, heads_tab=4)
_edge_l2 = _make_edge_kernel(n_pass=1, split_edges=True, heads_tab=1)


def _mid_body(ov_ref, od_ref, b1_ref, w2_ref, b2_ref, hl_ref, hr_ref):
    o = ov_ref[...]
    d = od_ref[...]
    hh = o / (d[:, :, 0:1] + 1e-16) + b1_ref[...][:, None, :]
    hh = jnp.maximum(hh, 0.01 * hh)
    acc = jnp.zeros((BM, 2 * HID), jnp.float32)
    for q in range(HEADS):
        acc = acc + jnp.dot(hh[q], w2_ref[q], preferred_element_type=jnp.float32)
    acc = acc + b2_ref[...]
    hl_ref[...] = acc[:, :HID]
    hr_ref[...] = acc[:, HID:]


def _mid(ov, od, b1, w2, b2):
    return pl.pallas_call(
        _mid_body,
        grid=(NP // BM,),
        in_specs=[
            pl.BlockSpec((HEADS, BM, HID), lambda m: (0, m, 0)),
            pl.BlockSpec((HEADS, BM, 16), lambda m: (0, m, 0)),
            pl.BlockSpec((HEADS, HID), lambda m: (0, 0)),
            pl.BlockSpec((HEADS, HID, 2 * HID), lambda m: (0, 0, 0)),
            pl.BlockSpec((1, 2 * HID), lambda m: (0, 0)),
        ],
        out_specs=[
            pl.BlockSpec((BM, HID), lambda m: (m, 0)),
            pl.BlockSpec((BM, HID), lambda m: (m, 0)),
        ],
        out_shape=[jax.ShapeDtypeStruct((NP, HID), jnp.float32),
                   jax.ShapeDtypeStruct((NP, HID), jnp.float32)],
    )(ov, od, b1, w2, b2)


def _fin_body(o2_ref, d2_ref, b2_ref, id_ref, wp1_ref, bp1_ref, wp2_ref,
              bp2_ref, hf_ref, lg_ref):
    o = o2_ref[0] + o2_ref[1]
    d = d2_ref[0, :, 0:1] + d2_ref[1, :, 0:1]
    h2 = o / (d + 1e-16) + b2_ref[...]
    hf = h2 + id_ref[...]
    hf_ref[...] = hf
    z = jnp.dot(hf, wp1_ref[...], preferred_element_type=jnp.float32) + bp1_ref[...]
    z = jnp.maximum(z, 0.01 * z)
    lg_ref[...] = jnp.dot(z, wp2_ref[...], preferred_element_type=jnp.float32) + bp2_ref[...]


def _fin(o2, d2, b2, ident, wp1, bp1, wp2, bp2):
    return pl.pallas_call(
        _fin_body,
        grid=(NP // BM,),
        in_specs=[
            pl.BlockSpec((2, BM, HID), lambda m: (0, m, 0)),
            pl.BlockSpec((2, BM, 16), lambda m: (0, m, 0)),
            pl.BlockSpec((1, HID), lambda m: (0, 0)),
            pl.BlockSpec((BM, HID), lambda m: (m, 0)),
            pl.BlockSpec((HID, 32), lambda m: (0, 0)),
            pl.BlockSpec((1, 32), lambda m: (0, 0)),
            pl.BlockSpec((32, 8), lambda m: (0, 0)),
            pl.BlockSpec((1, 8), lambda m: (0, 0)),
        ],
        out_specs=[
            pl.BlockSpec((BM, HID), lambda m: (m, 0)),
            pl.BlockSpec((BM, 8), lambda m: (m, 0)),
        ],
        out_shape=[jax.ShapeDtypeStruct((NP, HID), jnp.float32),
                   jax.ShapeDtypeStruct((NP, 8), jnp.float32)],
    )(o2, d2, b2, ident, wp1, bp1, wp2, bp2)


def kernel(x, edge_index, W1l, b1l, W1r, b1r, att1, bias1, W2l, b2l, W2r,
           b2r, att2, bias2, Ws, bs, Wp1, bp1, Wp2, bp2):
    xp = jnp.zeros((NP, F), jnp.float32).at[:N].set(x)
    loop = jnp.arange(N, dtype=jnp.int32)
    pad = jnp.full((E_PAD - E_TOT,), N, jnp.int32)
    srcE = jnp.concatenate([edge_index[0], loop, pad])
    dstE = jnp.concatenate([edge_index[1], loop, pad])

    Wbig = jnp.concatenate([W1l, W1r, Ws], axis=1)
    Wbig = Wbig.reshape(F, 9, HID).transpose(1, 0, 2)
    bbig = jnp.concatenate([b1l, b1r, bs]).reshape(9, 1, HID)
    OUT = _proj(xp, Wbig, bbig)
    XL = OUT[0:4].reshape(4 * NP, HID)
    XR = OUT[4:8].reshape(4 * NP, HID)
    ident = OUT[8]

    zv = jnp.zeros((2 * NP, HID), jnp.float32)
    zd = jnp.zeros((2 * NP, 16), jnp.float32)
    ov1, od1 = _edge_l1(XL, XR, srcE, dstE, att1, zv, zd)

    w2 = jnp.concatenate([W2l, W2r], axis=1).reshape(HEADS, HID, 2 * HID)
    b2cat = jnp.concatenate([b2l, b2r]).reshape(1, 2 * HID)
    HLt, HRt = _mid(ov1.reshape(HEADS, NP, HID),
                    od1.reshape(HEADS, NP, 16),
                    bias1.reshape(HEADS, HID), w2, b2cat)

    ov2, od2 = _edge_l2(HLt, HRt, srcE, dstE, att2, zv[:NP], zd[:NP])

    wp2p = jnp.zeros((32, 8), jnp.float32).at[:, 0:1].set(Wp2)
    bp2p = jnp.zeros((1, 8), jnp.float32).at[0, 0].set(bp2[0])
    HF, LG = _fin(ov2.reshape(2, NP, HID), od2.reshape(2, NP, 16),
                  bias2.reshape(1, HID), ident, Wp1, bp1.reshape(1, 32),
                  wp2p, bp2p)
    return LG[:N, 0:1], HF[:N]


# speedup vs baseline: 1.1819x; 1.1819x over previous
import functools

import jax
import jax.numpy as jnp
from jax import lax
from jax.experimental import pallas as pl
from jax.experimental.pallas import tpu as pltpu
from jax.experimental.pallas import tpu_sc as plsc

N = 10000
NP = 10240
F = 128
HID = 64
HEADS = 4
E_TOT = 330000
C = 64
E_PAD = 335872
BM = 1024


def _proj_body(x_ref, w_ref, b_ref, o_ref):
    o_ref[...] = (jnp.dot(x_ref[...], w_ref[0],
                          preferred_element_type=jnp.float32) + b_ref[0])[None]


def _proj(xp, Wbig, bbig):
    return pl.pallas_call(
        _proj_body,
        grid=(9, NP // BM),
        in_specs=[
            pl.BlockSpec((BM, F), lambda j, m: (m, 0)),
            pl.BlockSpec((1, F, HID), lambda j, m: (j, 0, 0)),
            pl.BlockSpec((1, 1, HID), lambda j, m: (j, 0, 0)),
        ],
        out_specs=pl.BlockSpec((1, BM, HID), lambda j, m: (j, m, 0)),
        out_shape=jax.ShapeDtypeStruct((9, NP, HID), jnp.float32),
    )(xp, Wbig, bbig)


def _make_edge_kernel(n_pass, split_edges, heads_tab, interpret=False):
    ACC = n_pass * NP
    ept = E_PAD // (32 if split_edges else 16)
    n_chunks = ept // C
    wrows = ACC // 16
    mesh = plsc.VectorSubcoreMesh(core_axis_name="c", subcore_axis_name="s")

    @functools.partial(
        pl.kernel,
        out_type=(jax.ShapeDtypeStruct((2 * ACC, HID), jnp.float32),
                  jax.ShapeDtypeStruct((2 * ACC, 16), jnp.float32)),
        mesh=mesh,
        scratch_types=[
            pltpu.VMEM_SHARED((ACC, HID), jnp.float32),
            pltpu.VMEM_SHARED((ACC, 16), jnp.float32),
            pltpu.VMEM((heads_tab, HID), jnp.float32),
            pltpu.VMEM((C,), jnp.int32),
            pltpu.VMEM((C,), jnp.int32),
            pltpu.VMEM((C,), jnp.int32),
            pltpu.VMEM((C,), jnp.int32),
            pltpu.VMEM((C,), jnp.int32),
            pltpu.VMEM((C,), jnp.int32),
            pltpu.VMEM((C,), jnp.int32),
            pltpu.VMEM((C,), jnp.int32),
            pltpu.VMEM((C, HID), jnp.float32),
            pltpu.VMEM((C, HID), jnp.float32),
            pltpu.VMEM((C, HID), jnp.float32),
            pltpu.VMEM((C, HID), jnp.float32),
            pltpu.VMEM((C, HID), jnp.float32),
            pltpu.VMEM((C, 16), jnp.float32),
            pltpu.SemaphoreType.DMA,
            pltpu.SemaphoreType.DMA,
            pltpu.SemaphoreType.DMA,
            pltpu.SemaphoreType.DMA,
        ],
        compiler_params=pltpu.CompilerParams(needs_layout_passes=False,
                                             use_tc_tiling_on_sc=False),
        interpret=interpret,
    )
    def k(xl, xr, srcE, dstE, attH, zv, zd, outv, outd,
          accv, accd, att_vm, src_i, dst_i,
          srcg0, dstg0, dsta0, srcg1, dstg1, dsta1,
          xls0, xrd0, xls1, xrd1, vals, exb,
          sa0, sb0, sa1, sb1):
        c = lax.axis_index("c")
        s = lax.axis_index("s")
        pltpu.sync_copy(zv.at[pl.ds(s * wrows, wrows)],
                        accv.at[pl.ds(s * wrows, wrows)])
        pltpu.sync_copy(zd.at[pl.ds(s * wrows, wrows)],
                        accd.at[pl.ds(s * wrows, wrows)])
        pltpu.sync_copy(attH, att_vm)
        plsc.subcore_barrier()
        lane = lax.iota(jnp.int32, 16)
        mask0 = lane == 0
        bufs = ((srcg0, dstg0, dsta0, xls0, xrd0, sa0, sb0),
                (srcg1, dstg1, dsta1, xls1, xrd1, sa1, sb1))

        for p in range(n_pass):
            if split_edges:
                tile_base = (c * 16 + s) * ept
                g_off = jnp.int32(0)
                a_off = jnp.int32(0)
                h_idx = 0
            else:
                tile_base = s * ept
                h_idx = 2 * c + p
                g_off = h_idx * NP
                a_off = jnp.int32(p * NP)
            att_q = [att_vm[h_idx, pl.ds(q * 16, 16)] for q in range(HID // 16)]

            def load(kk, b):
                srcg, dstg, dsta, xls, xrd, sa, sb = bufs[b]
                eb = lax.rem(tile_base + kk * C, jnp.int32(E_PAD))
                pltpu.sync_copy(srcE.at[pl.ds(eb, C)], src_i)
                pltpu.sync_copy(dstE.at[pl.ds(eb, C)], dst_i)

                def shift(i, _):
                    sl = pl.ds(i * 16, 16)
                    srcg[sl] = src_i[sl] + g_off
                    dstg[sl] = dst_i[sl] + g_off
                    dsta[sl] = dst_i[sl] + a_off
                    return 0
                lax.fori_loop(0, C // 16, shift, 0)
                pltpu.async_copy(xl.at[srcg], xls, sa)
                pltpu.async_copy(xr.at[dstg], xrd, sb)

            def wait_bufs(b):
                srcg, dstg, dsta, xls, xrd, sa, sb = bufs[b]
                pltpu.make_async_copy(xl.at[srcg], xls, sa).wait()
                pltpu.make_async_copy(xr.at[dstg], xrd, sb).wait()

            def compute(b):
                srcg, dstg, dsta, xls, xrd, sa, sb = bufs[b]

                def edge(e, _):
                    acc = jnp.zeros((16,), jnp.float32)
                    rows = []
                    for q in range(HID // 16):
                        sv = xls[e, pl.ds(q * 16, 16)]
                        rv = xrd[e, pl.ds(q * 16, 16)]
                        t = sv + rv
                        t = jnp.maximum(t, 0.2 * t)
                        acc = acc + t * att_q[q]
                        rows.append(sv)
                    logit = jnp.sum(acc)
                    exv = jnp.exp(jnp.full((16,), logit, jnp.float32))
                    for q in range(HID // 16):
                        vals[e, pl.ds(q * 16, 16)] = rows[q] * exv
                    exb[e, :] = jnp.where(mask0, exv, 0.0)
                    return 0
                lax.fori_loop(0, C, edge, 0)
                pltpu.sync_copy(vals, accv.at[dsta], add=True)
                pltpu.sync_copy(exb, accd.at[dsta], add=True)

            load(jnp.int32(0), 0)

            def pair(p2, _):
                k0 = p2 * 2
                load(k0 + 1, 1)
                wait_bufs(0)
                compute(0)
                load(k0 + 2, 0)
                wait_bufs(1)
                compute(1)
                return 0
            lax.fori_loop(0, n_chunks // 2, pair, 0)
            wait_bufs(0)

        plsc.subcore_barrier()
        ob = c * ACC + s * wrows
        pltpu.sync_copy(accv.at[pl.ds(s * wrows, wrows)],
                        outv.at[pl.ds(ob, wrows)])
        pltpu.sync_copy(accd.at[pl.ds(s * wrows, wrows)],
                        outd.at[pl.ds(ob, wrows)])

    return k


_edge_l1 = _make_edge_kernel(n_pass=2, split_edges=False, heads_tab=4)
_edge_l2 = _make_edge_kernel(n_pass=1, split_edges=True, heads_tab=1)


def _mid_body(ov_ref, od_ref, b1_ref, w2_ref, b2_ref, hl_ref, hr_ref):
    o = ov_ref[...]
    d = od_ref[...]
    hh = o / (d[:, :, 0:1] + 1e-16) + b1_ref[...][:, None, :]
    hh = jnp.maximum(hh, 0.01 * hh)
    acc = jnp.zeros((BM, 2 * HID), jnp.float32)
    for q in range(HEADS):
        acc = acc + jnp.dot(hh[q], w2_ref[q], preferred_element_type=jnp.float32)
    acc = acc + b2_ref[...]
    hl_ref[...] = acc[:, :HID]
    hr_ref[...] = acc[:, HID:]


def _mid(ov, od, b1, w2, b2):
    return pl.pallas_call(
        _mid_body,
        grid=(NP // BM,),
        in_specs=[
            pl.BlockSpec((HEADS, BM, HID), lambda m: (0, m, 0)),
            pl.BlockSpec((HEADS, BM, 16), lambda m: (0, m, 0)),
            pl.BlockSpec((HEADS, HID), lambda m: (0, 0)),
            pl.BlockSpec((HEADS, HID, 2 * HID), lambda m: (0, 0, 0)),
            pl.BlockSpec((1, 2 * HID), lambda m: (0, 0)),
        ],
        out_specs=[
            pl.BlockSpec((BM, HID), lambda m: (m, 0)),
            pl.BlockSpec((BM, HID), lambda m: (m, 0)),
        ],
        out_shape=[jax.ShapeDtypeStruct((NP, HID), jnp.float32),
                   jax.ShapeDtypeStruct((NP, HID), jnp.float32)],
    )(ov, od, b1, w2, b2)


def _fin_body(o2_ref, d2_ref, b2_ref, id_ref, wp1_ref, bp1_ref, wp2_ref,
              bp2_ref, hf_ref, lg_ref):
    o = o2_ref[0] + o2_ref[1]
    d = d2_ref[0, :, 0:1] + d2_ref[1, :, 0:1]
    h2 = o / (d + 1e-16) + b2_ref[...]
    hf = h2 + id_ref[...]
    hf_ref[...] = hf
    z = jnp.dot(hf, wp1_ref[...], preferred_element_type=jnp.float32) + bp1_ref[...]
    z = jnp.maximum(z, 0.01 * z)
    lg_ref[...] = jnp.dot(z, wp2_ref[...], preferred_element_type=jnp.float32) + bp2_ref[...]


def _fin(o2, d2, b2, ident, wp1, bp1, wp2, bp2):
    return pl.pallas_call(
        _fin_body,
        grid=(NP // BM,),
        in_specs=[
            pl.BlockSpec((2, BM, HID), lambda m: (0, m, 0)),
            pl.BlockSpec((2, BM, 16), lambda m: (0, m, 0)),
            pl.BlockSpec((1, HID), lambda m: (0, 0)),
            pl.BlockSpec((BM, HID), lambda m: (m, 0)),
            pl.BlockSpec((HID, 32), lambda m: (0, 0)),
            pl.BlockSpec((1, 32), lambda m: (0, 0)),
            pl.BlockSpec((32, 8), lambda m: (0, 0)),
            pl.BlockSpec((1, 8), lambda m: (0, 0)),
        ],
        out_specs=[
            pl.BlockSpec((BM, HID), lambda m: (m, 0)),
            pl.BlockSpec((BM, 8), lambda m: (m, 0)),
        ],
        out_shape=[jax.ShapeDtypeStruct((NP, HID), jnp.float32),
                   jax.ShapeDtypeStruct((NP, 8), jnp.float32)],
    )(o2, d2, b2, ident, wp1, bp1, wp2, bp2)


def kernel(x, edge_index, W1l, b1l, W1r, b1r, att1, bias1, W2l, b2l, W2r,
           b2r, att2, bias2, Ws, bs, Wp1, bp1, Wp2, bp2):
    xp = jnp.zeros((NP, F), jnp.float32).at[:N].set(x)
    loop = jnp.arange(N, dtype=jnp.int32)
    pad = jnp.full((E_PAD - E_TOT,), N, jnp.int32)
    srcE = jnp.concatenate([edge_index[0], loop, pad])
    dstE = jnp.concatenate([edge_index[1], loop, pad])

    Wbig = jnp.concatenate([W1l, W1r, Ws], axis=1)
    Wbig = Wbig.reshape(F, 9, HID).transpose(1, 0, 2)
    bbig = jnp.concatenate([b1l, b1r, bs]).reshape(9, 1, HID)
    OUT = _proj(xp, Wbig, bbig)
    XL = OUT[0:4].reshape(4 * NP, HID)
    XR = OUT[4:8].reshape(4 * NP, HID)
    ident = OUT[8]

    zv = jnp.zeros((2 * NP, HID), jnp.float32)
    zd = jnp.zeros((2 * NP, 16), jnp.float32)
    ov1, od1 = _edge_l1(XL, XR, srcE, dstE, att1, zv, zd)

    w2 = jnp.concatenate([W2l, W2r], axis=1).reshape(HEADS, HID, 2 * HID)
    b2cat = jnp.concatenate([b2l, b2r]).reshape(1, 2 * HID)
    HLt, HRt = _mid(ov1.reshape(HEADS, NP, HID),
                    od1.reshape(HEADS, NP, 16),
                    bias1.reshape(HEADS, HID), w2, b2cat)

    ov2, od2 = _edge_l2(HLt, HRt, srcE, dstE, att2, zv[:NP], zd[:NP])

    wp2p = jnp.zeros((32, 8), jnp.float32).at[:, 0:1].set(Wp2)
    bp2p = jnp.zeros((1, 8), jnp.float32).at[0, 0].set(bp2[0])
    HF, LG = _fin(ov2.reshape(2, NP, HID), od2.reshape(2, NP, 16),
                  bias2.reshape(1, HID), ident, Wp1, bp1.reshape(1, 32),
                  wp2p, bp2p)
    return LG[:N, 0:1], HF[:N]


# speedup vs baseline: 1.3977x; 1.1826x over previous
import functools

import jax
import jax.numpy as jnp
from jax import lax
from jax.experimental import pallas as pl
from jax.experimental.pallas import tpu as pltpu
from jax.experimental.pallas import tpu_sc as plsc

N = 10000
NP = 10240
F = 128
HID = 64
HEADS = 4
E_TOT = 330000
C = 64
E_PAD = 344064
IB = 512
BM = 1024


def _proj_body(x_ref, w_ref, b_ref, o_ref):
    o_ref[...] = (jnp.dot(x_ref[...], w_ref[0],
                          preferred_element_type=jnp.float32) + b_ref[0])[None]


def _proj(xp, Wbig, bbig):
    return pl.pallas_call(
        _proj_body,
        grid=(9, NP // BM),
        in_specs=[
            pl.BlockSpec((BM, F), lambda j, m: (m, 0)),
            pl.BlockSpec((1, F, HID), lambda j, m: (j, 0, 0)),
            pl.BlockSpec((1, 1, HID), lambda j, m: (j, 0, 0)),
        ],
        out_specs=pl.BlockSpec((1, BM, HID), lambda j, m: (j, m, 0)),
        out_shape=jax.ShapeDtypeStruct((9, NP, HID), jnp.float32),
    )(xp, Wbig, bbig)


def _make_edge_kernel(n_pass, split_edges, heads_tab, interpret=False):
    ACC = n_pass * NP
    ept = E_PAD // (32 if split_edges else 16)
    n_chunks = ept // C
    wrows = ACC // 16
    mesh = plsc.VectorSubcoreMesh(core_axis_name="c", subcore_axis_name="s")

    @functools.partial(
        pl.kernel,
        out_type=(jax.ShapeDtypeStruct((2 * ACC, HID), jnp.float32),
                  jax.ShapeDtypeStruct((2 * ACC, 16), jnp.float32)),
        mesh=mesh,
        scratch_types=[
            pltpu.VMEM_SHARED((ACC, HID), jnp.float32),
            pltpu.VMEM_SHARED((ACC, 16), jnp.float32),
            pltpu.VMEM((heads_tab, HID), jnp.float32),
            pltpu.VMEM((IB,), jnp.int32),
            pltpu.VMEM((IB,), jnp.int32),
            pltpu.VMEM((C,), jnp.int32),
            pltpu.VMEM((C,), jnp.int32),
            pltpu.VMEM((C,), jnp.int32),
            pltpu.VMEM((C,), jnp.int32),
            pltpu.VMEM((C,), jnp.int32),
            pltpu.VMEM((C,), jnp.int32),
            pltpu.VMEM((C, HID), jnp.float32),
            pltpu.VMEM((C, HID), jnp.float32),
            pltpu.VMEM((C, HID), jnp.float32),
            pltpu.VMEM((C, HID), jnp.float32),
            pltpu.VMEM((C, HID), jnp.float32),
            pltpu.VMEM((C, 16), jnp.float32),
            pltpu.SemaphoreType.DMA,
            pltpu.SemaphoreType.DMA,
            pltpu.SemaphoreType.DMA,
            pltpu.SemaphoreType.DMA,
        ],
        compiler_params=pltpu.CompilerParams(needs_layout_passes=False,
                                             use_tc_tiling_on_sc=False),
        interpret=interpret,
    )
    def k(xl, xr, srcE, dstE, attH, zv, zd, outv, outd,
          accv, accd, att_vm, src_i, dst_i,
          srcg0, dstg0, dsta0, srcg1, dstg1, dsta1,
          xls0, xrd0, xls1, xrd1, vals, exb,
          sa0, sb0, sa1, sb1):
        c = lax.axis_index("c")
        s = lax.axis_index("s")
        pltpu.sync_copy(zv.at[pl.ds(s * wrows, wrows)],
                        accv.at[pl.ds(s * wrows, wrows)])
        pltpu.sync_copy(zd.at[pl.ds(s * wrows, wrows)],
                        accd.at[pl.ds(s * wrows, wrows)])
        pltpu.sync_copy(attH, att_vm)
        plsc.subcore_barrier()
        lane = lax.iota(jnp.int32, 16)
        mask0 = lane == 0
        bufs = ((srcg0, dstg0, dsta0, xls0, xrd0, sa0, sb0),
                (srcg1, dstg1, dsta1, xls1, xrd1, sa1, sb1))

        for p in range(n_pass):
            if split_edges:
                tile_base = (c * 16 + s) * ept
                g_off = jnp.int32(0)
                a_off = jnp.int32(0)
                h_idx = 0
            else:
                tile_base = s * ept
                h_idx = 2 * c + p
                g_off = h_idx * NP
                a_off = jnp.int32(p * NP)
            att_q = [att_vm[h_idx, pl.ds(q * 16, 16)] for q in range(HID // 16)]

            def load(kk, b):
                srcg, dstg, dsta, xls, xrd, sa, sb = bufs[b]
                blk = lax.rem(kk, jnp.int32(IB // C))

                @pl.when(blk == 0)
                def _():
                    eb = lax.rem(tile_base + kk * C, jnp.int32(E_PAD))
                    pltpu.sync_copy(srcE.at[pl.ds(eb, IB)], src_i)
                    pltpu.sync_copy(dstE.at[pl.ds(eb, IB)], dst_i)

                off = blk * C

                def shift(i, _):
                    so = pl.ds(off + i * 16, 16)
                    sl = pl.ds(i * 16, 16)
                    srcg[sl] = src_i[so] + g_off
                    dstg[sl] = dst_i[so] + g_off
                    dsta[sl] = dst_i[so] + a_off
                    return 0
                lax.fori_loop(0, C // 16, shift, 0)
                pltpu.async_copy(xl.at[srcg], xls, sa)
                pltpu.async_copy(xr.at[dstg], xrd, sb)

            def wait_bufs(b):
                srcg, dstg, dsta, xls, xrd, sa, sb = bufs[b]
                pltpu.make_async_copy(xl.at[srcg], xls, sa).wait()
                pltpu.make_async_copy(xr.at[dstg], xrd, sb).wait()

            def compute(b):
                srcg, dstg, dsta, xls, xrd, sa, sb = bufs[b]

                def edge(e, _):
                    acc = jnp.zeros((16,), jnp.float32)
                    rows = []
                    for q in range(HID // 16):
                        sv = xls[e, pl.ds(q * 16, 16)]
                        rv = xrd[e, pl.ds(q * 16, 16)]
                        t = sv + rv
                        t = jnp.maximum(t, 0.2 * t)
                        acc = acc + t * att_q[q]
                        rows.append(sv)
                    logit = jnp.sum(acc)
                    exv = jnp.exp(jnp.full((16,), logit, jnp.float32))
                    for q in range(HID // 16):
                        vals[e, pl.ds(q * 16, 16)] = rows[q] * exv
                    exb[e, :] = jnp.where(mask0, exv, 0.0)
                    return 0
                lax.fori_loop(0, C, edge, 0)
                pltpu.sync_copy(vals, accv.at[dsta], add=True)
                pltpu.sync_copy(exb, accd.at[dsta], add=True)

            load(jnp.int32(0), 0)

            def pair(p2, _):
                k0 = p2 * 2
                load(k0 + 1, 1)
                wait_bufs(0)
                compute(0)
                load(k0 + 2, 0)
                wait_bufs(1)
                compute(1)
                return 0
            lax.fori_loop(0, n_chunks // 2, pair, 0)
            wait_bufs(0)

        plsc.subcore_barrier()
        ob = c * ACC + s * wrows
        pltpu.sync_copy(accv.at[pl.ds(s * wrows, wrows)],
                        outv.at[pl.ds(ob, wrows)])
        pltpu.sync_copy(accd.at[pl.ds(s * wrows, wrows)],
                        outd.at[pl.ds(ob, wrows)])

    return k


_edge_l1 = _make_edge_kernel(n_pass=2, split_edges=False, heads_tab=4)
_edge_l2 = _make_edge_kernel(n_pass=1, split_edges=True, heads_tab=1)


def _mid_body(ov_ref, od_ref, b1_ref, w2_ref, b2_ref, hl_ref, hr_ref):
    o = ov_ref[...]
    d = od_ref[...]
    hh = o / (d[:, :, 0:1] + 1e-16) + b1_ref[...][:, None, :]
    hh = jnp.maximum(hh, 0.01 * hh)
    acc = jnp.zeros((BM, 2 * HID), jnp.float32)
    for q in range(HEADS):
        acc = acc + jnp.dot(hh[q], w2_ref[q], preferred_element_type=jnp.float32)
    acc = acc + b2_ref[...]
    hl_ref[...] = acc[:, :HID]
    hr_ref[...] = acc[:, HID:]


def _mid(ov, od, b1, w2, b2):
    return pl.pallas_call(
        _mid_body,
        grid=(NP // BM,),
        in_specs=[
            pl.BlockSpec((HEADS, BM, HID), lambda m: (0, m, 0)),
            pl.BlockSpec((HEADS, BM, 16), lambda m: (0, m, 0)),
            pl.BlockSpec((HEADS, HID), lambda m: (0, 0)),
            pl.BlockSpec((HEADS, HID, 2 * HID), lambda m: (0, 0, 0)),
            pl.BlockSpec((1, 2 * HID), lambda m: (0, 0)),
        ],
        out_specs=[
            pl.BlockSpec((BM, HID), lambda m: (m, 0)),
            pl.BlockSpec((BM, HID), lambda m: (m, 0)),
        ],
        out_shape=[jax.ShapeDtypeStruct((NP, HID), jnp.float32),
                   jax.ShapeDtypeStruct((NP, HID), jnp.float32)],
    )(ov, od, b1, w2, b2)


def _fin_body(o2_ref, d2_ref, b2_ref, id_ref, wp1_ref, bp1_ref, wp2_ref,
              bp2_ref, hf_ref, lg_ref):
    o = o2_ref[0] + o2_ref[1]
    d = d2_ref[0, :, 0:1] + d2_ref[1, :, 0:1]
    h2 = o / (d + 1e-16) + b2_ref[...]
    hf = h2 + id_ref[...]
    hf_ref[...] = hf
    z = jnp.dot(hf, wp1_ref[...], preferred_element_type=jnp.float32) + bp1_ref[...]
    z = jnp.maximum(z, 0.01 * z)
    lg_ref[...] = jnp.dot(z, wp2_ref[...], preferred_element_type=jnp.float32) + bp2_ref[...]


def _fin(o2, d2, b2, ident, wp1, bp1, wp2, bp2):
    return pl.pallas_call(
        _fin_body,
        grid=(NP // BM,),
        in_specs=[
            pl.BlockSpec((2, BM, HID), lambda m: (0, m, 0)),
            pl.BlockSpec((2, BM, 16), lambda m: (0, m, 0)),
            pl.BlockSpec((1, HID), lambda m: (0, 0)),
            pl.BlockSpec((BM, HID), lambda m: (m, 0)),
            pl.BlockSpec((HID, 32), lambda m: (0, 0)),
            pl.BlockSpec((1, 32), lambda m: (0, 0)),
            pl.BlockSpec((32, 8), lambda m: (0, 0)),
            pl.BlockSpec((1, 8), lambda m: (0, 0)),
        ],
        out_specs=[
            pl.BlockSpec((BM, HID), lambda m: (m, 0)),
            pl.BlockSpec((BM, 8), lambda m: (m, 0)),
        ],
        out_shape=[jax.ShapeDtypeStruct((NP, HID), jnp.float32),
                   jax.ShapeDtypeStruct((NP, 8), jnp.float32)],
    )(o2, d2, b2, ident, wp1, bp1, wp2, bp2)


def kernel(x, edge_index, W1l, b1l, W1r, b1r, att1, bias1, W2l, b2l, W2r,
           b2r, att2, bias2, Ws, bs, Wp1, bp1, Wp2, bp2):
    xp = jnp.zeros((NP, F), jnp.float32).at[:N].set(x)
    loop = jnp.arange(N, dtype=jnp.int32)
    pad = jnp.full((E_PAD - E_TOT,), N, jnp.int32)
    srcE = jnp.concatenate([edge_index[0], loop, pad])
    dstE = jnp.concatenate([edge_index[1], loop, pad])

    Wbig = jnp.concatenate([W1l, W1r, Ws], axis=1)
    Wbig = Wbig.reshape(F, 9, HID).transpose(1, 0, 2)
    bbig = jnp.concatenate([b1l, b1r, bs]).reshape(9, 1, HID)
    OUT = _proj(xp, Wbig, bbig)
    XL = OUT[0:4].reshape(4 * NP, HID)
    XR = OUT[4:8].reshape(4 * NP, HID)
    ident = OUT[8]

    zv = jnp.zeros((2 * NP, HID), jnp.float32)
    zd = jnp.zeros((2 * NP, 16), jnp.float32)
    ov1, od1 = _edge_l1(XL, XR, srcE, dstE, att1, zv, zd)

    w2 = jnp.concatenate([W2l, W2r], axis=1).reshape(HEADS, HID, 2 * HID)
    b2cat = jnp.concatenate([b2l, b2r]).reshape(1, 2 * HID)
    HLt, HRt = _mid(ov1.reshape(HEADS, NP, HID),
                    od1.reshape(HEADS, NP, 16),
                    bias1.reshape(HEADS, HID), w2, b2cat)

    ov2, od2 = _edge_l2(HLt, HRt, srcE, dstE, att2, zv[:NP], zd[:NP])

    wp2p = jnp.zeros((32, 8), jnp.float32).at[:, 0:1].set(Wp2)
    bp2p = jnp.zeros((1, 8), jnp.float32).at[0, 0].set(bp2[0])
    HF, LG = _fin(ov2.reshape(2, NP, HID), od2.reshape(2, NP, 16),
                  bias2.reshape(1, HID), ident, Wp1, bp1.reshape(1, 32),
                  wp2p, bp2p)
    return LG[:N, 0:1], HF[:N]
